# Initial kernel scaffold; baseline (speedup 1.0000x reference)
import jax
import jax.numpy as jnp
from jax.experimental import pallas as pl


def kernel(pos, x, p_w, p_b, W1, b1, W2, b2, W3, b3):
    raise NotImplementedError("write your pallas kernel here")


# speedup vs baseline: 1.5466x; 1.5466x over previous
import functools

import jax
import jax.numpy as jnp
from jax.experimental import pallas as pl
from jax.experimental.pallas import tpu as pltpu

_BLK = 8000


def _fused_pass(y_ref, pos_ref, x_ref, w1_ref, b1_ref, w2_ref, b2_ref,
                w3_ref, b3_ref, g_ref, padj_ref):
    y = y_ref[...]
    g = jax.nn.sigmoid(y)
    xg = x_ref[...] * g
    h = jnp.maximum(jnp.dot(xg, w1_ref[...],
                            preferred_element_type=jnp.float32) + b1_ref[...], 0.0)
    h = jnp.maximum(jnp.dot(h, w2_ref[...],
                            preferred_element_type=jnp.float32) + b2_ref[...], 0.0)
    delta = jnp.dot(h, w3_ref[...], preferred_element_type=jnp.float32) + b3_ref[...]
    g_ref[...] = g
    padj_ref[...] = pos_ref[...] + delta


def kernel(pos, x, p_w, p_b, W1, b1, W2, b2, W3, b3):
    B, N, D = x.shape
    k = int(N * 0.5)
    BN = B * N

    y = (x @ p_w + p_b) / jnp.linalg.norm(p_w)
    scores = y[..., 0]

    yf = y.reshape(BN, 1)
    xf = x.reshape(BN, D)
    posf = pos.reshape(BN, 3)

    grid = (BN // _BLK,)
    row_spec = lambda w: pl.BlockSpec((_BLK, w), lambda i: (i, 0))
    full = lambda shape: pl.BlockSpec(shape, lambda i: tuple(0 for _ in shape))

    g_all, padj = pl.pallas_call(
        _fused_pass,
        grid=grid,
        in_specs=[
            row_spec(1),
            row_spec(3),
            row_spec(D),
            full((D, D // 2)), full((D // 2,)),
            full((D // 2, D // 4)), full((D // 4,)),
            full((D // 4, 3)), full((3,)),
        ],
        out_specs=[row_spec(1), row_spec(3)],
        out_shape=[
            jax.ShapeDtypeStruct((BN, 1), jnp.float32),
            jax.ShapeDtypeStruct((BN, 3), jnp.float32),
        ],
    )(yf, posf, xf, W1, b1, W2, b2, W3, b3)

    g_all = g_all.reshape(B, N)
    padj = padj.reshape(B, N, 3)

    _, idx = jax.lax.top_k(scores, k)
    pos_out = jnp.take_along_axis(padj, idx[:, :, None], axis=1)
    g_sel = jnp.take_along_axis(g_all, idx, axis=1)
    x_sel = jnp.take_along_axis(x, idx[:, :, None], axis=1) * g_sel[:, :, None]
    return idx, pos_out, x_sel


# speedup vs baseline: 1.8002x; 1.1640x over previous
import functools

import jax
import jax.numpy as jnp
from jax.experimental import pallas as pl
from jax.experimental.pallas import tpu as pltpu

_BLK = 8000


def _fused_pass(pos_ref, x_ref, pw_ref, pb_ref, nrm_ref, w1_ref, b1_ref,
                w2_ref, b2_ref, w3_ref, b3_ref, y_ref, g_ref, padj_ref):
    s = jnp.dot(x_ref[...], pw_ref[...],
                preferred_element_type=jnp.float32) + pb_ref[...]
    y = s / nrm_ref[...]
    g = jax.nn.sigmoid(y)
    xg = x_ref[...] * g
    h = jnp.maximum(jnp.dot(xg, w1_ref[...],
                            preferred_element_type=jnp.float32) + b1_ref[...], 0.0)
    h = jnp.maximum(jnp.dot(h, w2_ref[...],
                            preferred_element_type=jnp.float32) + b2_ref[...], 0.0)
    delta = jnp.dot(h, w3_ref[...], preferred_element_type=jnp.float32) + b3_ref[...]
    y_ref[...] = y
    g_ref[...] = g
    padj_ref[...] = pos_ref[...] + delta


def kernel(pos, x, p_w, p_b, W1, b1, W2, b2, W3, b3):
    B, N, D = x.shape
    k = int(N * 0.5)
    BN = B * N

    nrm = jnp.linalg.norm(p_w).reshape(1, 1)

    xf = x.reshape(BN, D)
    posf = pos.reshape(BN, 3)

    grid = (BN // _BLK,)
    row_spec = lambda w: pl.BlockSpec((_BLK, w), lambda i: (i, 0))
    full = lambda shape: pl.BlockSpec(shape, lambda i: tuple(0 for _ in shape))

    y_all, g_all, padj = pl.pallas_call(
        _fused_pass,
        grid=grid,
        in_specs=[
            row_spec(3),
            row_spec(D),
            full((D, 1)), full((1,)), full((1, 1)),
            full((D, D // 2)), full((D // 2,)),
            full((D // 2, D // 4)), full((D // 4,)),
            full((D // 4, 3)), full((3,)),
        ],
        out_specs=[row_spec(1), row_spec(1), row_spec(3)],
        out_shape=[
            jax.ShapeDtypeStruct((BN, 1), jnp.float32),
            jax.ShapeDtypeStruct((BN, 1), jnp.float32),
            jax.ShapeDtypeStruct((BN, 3), jnp.float32),
        ],
    )(posf, xf, p_w, p_b, nrm, W1, b1, W2, b2, W3, b3)

    scores = y_all.reshape(B, N)
    g_all = g_all.reshape(B, N)
    padj = padj.reshape(B, N, 3)

    _, idx = jax.lax.top_k(scores, k)
    pos_out = jnp.take_along_axis(padj, idx[:, :, None], axis=1)
    g_sel = jnp.take_along_axis(g_all, idx, axis=1)
    x_sel = jnp.take_along_axis(x, idx[:, :, None], axis=1) * g_sel[:, :, None]
    return idx, pos_out, x_sel
